# baseline (speedup 1.0000x reference)
import functools

import jax
import jax.numpy as jnp
from jax import lax
from jax.experimental import pallas as pl
from jax.experimental.pallas import tpu as pltpu
from jax.experimental.pallas import tpu_sc as plsc

NC = 2
NS = 16
NW = NC * NS
CHUNK = 128


def _deg_body(dst_hbm, ones_hbm, z_hbm, cnt_hbm, idx_v, ones_v, acc_sh):
    cid = lax.axis_index("c")
    sid = lax.axis_index("s")
    wid = cid * NS + sid
    cpt = dst_hbm.shape[1]
    rpt = acc_sh.shape[0] // NS

    pltpu.sync_copy(dst_hbm.at[wid], idx_v)
    pltpu.sync_copy(ones_hbm, ones_v)
    pltpu.sync_copy(z_hbm, acc_sh.at[pl.ds(sid * rpt, rpt)])
    plsc.subcore_barrier()

    @pl.loop(0, cpt)
    def _(j):
        pltpu.sync_copy(ones_v, acc_sh.at[idx_v.at[j]], add=True)

    plsc.subcore_barrier()
    pltpu.sync_copy(acc_sh.at[pl.ds(sid * rpt, rpt)],
                    cnt_hbm.at[cid, pl.ds(sid * rpt, rpt)])


NBUF = 2


PACK_BITS = 15


def _scatter_body(ya_hbm, yb_hbm, pk_hbm, z_hbm, out_hbm,
                  pkv, srcu, dstu, rows_v, acc_sh, gsem0, gsem1):
    cid = lax.axis_index("c")
    sid = lax.axis_index("s")
    wid = cid * NS + sid
    cpt = pk_hbm.shape[1]
    rpt = acc_sh.shape[0] // NS
    gsems = (gsem0, gsem1)
    mask = jnp.int32((1 << PACK_BITS) - 1)

    pltpu.sync_copy(pk_hbm.at[wid], pkv)
    pltpu.sync_copy(z_hbm, acc_sh.at[pl.ds(sid * rpt, rpt)])

    def unpack(j, b):
        for k in range(CHUNK // 16):
            p16 = pkv[j, pl.ds(k * 16, 16)]
            srcu[b, pl.ds(k * 16, 16)] = lax.shift_right_logical(
                p16, PACK_BITS)
            dstu[b, pl.ds(k * 16, 16)] = lax.bitwise_and(p16, mask)

    plsc.subcore_barrier()

    def run(y_hbm):
        for b in range(NBUF):
            unpack(b, b)
            pltpu.async_copy(y_hbm.at[srcu.at[b]], rows_v.at[b], gsems[b])

        @pl.loop(0, cpt, step=NBUF)
        def _(j0):
            for b in range(NBUF):
                j = j0 + b
                pltpu.make_async_copy(y_hbm.at[srcu.at[b]], rows_v.at[b],
                                      gsems[b]).wait()
                pltpu.sync_copy(rows_v.at[b], acc_sh.at[dstu.at[b]],
                                add=True)
                jn = jnp.minimum(j + NBUF, cpt - 1)
                unpack(jn, b)
                pltpu.async_copy(y_hbm.at[srcu.at[b]], rows_v.at[b],
                                 gsems[b])

        for b in range(NBUF):
            pltpu.make_async_copy(y_hbm.at[srcu.at[b]], rows_v.at[b],
                                  gsems[b]).wait()

    @pl.when(cid == 0)
    def _():
        run(ya_hbm)

    @pl.when(cid == 1)
    def _():
        run(yb_hbm)

    plsc.subcore_barrier()
    pltpu.sync_copy(acc_sh.at[pl.ds(sid * rpt, rpt)],
                    out_hbm.at[cid, pl.ds(sid * rpt, rpt)])


def _y_body(x_ref, w_ref, cnt_ref, y_ref, yb_ref):
    deg = cnt_ref[0, :, 0:1] + cnt_ref[1, :, 0:1] + 1.0
    dis = lax.rsqrt(deg)
    y = dis * jnp.dot(x_ref[...], w_ref[...],
                      preferred_element_type=jnp.float32)
    y_ref[...] = y
    yb_ref[...] = y


def _combine_body(p_ref, y_ref, cnt_ref, b_ref, o_ref):
    deg = cnt_ref[0, :, 0:1] + cnt_ref[1, :, 0:1] + 1.0
    dis = lax.rsqrt(deg)
    o_ref[...] = dis * (p_ref[0] + p_ref[1] + y_ref[...]) + b_ref[...]


def kernel(x, edge_index, weight, bias):
    n, d_in = x.shape
    d_out = weight.shape[1]
    e = edge_index.shape[1]

    src = edge_index[0].astype(jnp.int32)
    dst = edge_index[1].astype(jnp.int32)

    cpt = -(-e // (NW * CHUNK))
    cpt = -(-cpt // NBUF) * NBUF
    e_pad = NW * cpt * CHUNK
    pad = e_pad - e
    src_p = jnp.concatenate([src, jnp.zeros((pad,), jnp.int32)])
    dst_p = jnp.concatenate([dst, jnp.full((pad,), n, jnp.int32)])
    src3 = src_p.reshape(NW, cpt, CHUNK)
    dst3 = dst_p.reshape(NW, cpt, CHUNK)

    rpt = -(-(n + 1) // (NS * 8)) * 8
    r = rpt * NS

    ones128 = jnp.ones((CHUNK, 128), jnp.float32)
    z128 = jnp.zeros((rpt, d_out), jnp.float32)

    mesh = plsc.VectorSubcoreMesh(core_axis_name="c", subcore_axis_name="s")

    deg_k = pl.kernel(
        _deg_body,
        out_type=jax.ShapeDtypeStruct((NC, r, 128), jnp.float32),
        mesh=mesh,
        scratch_types=[
            pltpu.VMEM((cpt, CHUNK), jnp.int32),
            pltpu.VMEM((CHUNK, 128), jnp.float32),
            pltpu.VMEM_SHARED((r, 128), jnp.float32),
        ],
    )
    cnt = deg_k(dst3, ones128, z128)

    rb = 400
    grid = n // rb
    y = pl.pallas_call(
        _y_body,
        grid=(grid,),
        in_specs=[
            pl.BlockSpec((rb, d_in), lambda i: (i, 0)),
            pl.BlockSpec((d_in, d_out), lambda i: (0, 0)),
            pl.BlockSpec((NC, rb, 128), lambda i: (0, i, 0)),
        ],
        out_specs=[pl.BlockSpec((rb, d_out), lambda i: (i, 0)),
                   pl.BlockSpec((rb, d_out), lambda i: (i, 0))],
        out_shape=[jax.ShapeDtypeStruct((n, d_out), jnp.float32),
                   jax.ShapeDtypeStruct((n, d_out), jnp.float32)],
    )(x, weight, cnt)
    y, y_b = y

    packed3 = (src3 << PACK_BITS) | dst3
    scat_k = pl.kernel(
        _scatter_body,
        out_type=jax.ShapeDtypeStruct((NC, r, d_out), jnp.float32),
        mesh=mesh,
        scratch_types=[
            pltpu.VMEM((cpt, CHUNK), jnp.int32),
            pltpu.VMEM((NBUF, CHUNK), jnp.int32),
            pltpu.VMEM((NBUF, CHUNK), jnp.int32),
            pltpu.VMEM((NBUF, CHUNK, d_out), jnp.float32),
            pltpu.VMEM_SHARED((r, d_out), jnp.float32),
        ] + [pltpu.SemaphoreType.DMA] * NBUF,
    )
    partials = scat_k(y, y_b, packed3, z128)

    out = pl.pallas_call(
        _combine_body,
        grid=(grid,),
        in_specs=[
            pl.BlockSpec((NC, rb, d_out), lambda i: (0, i, 0)),
            pl.BlockSpec((rb, d_out), lambda i: (i, 0)),
            pl.BlockSpec((NC, rb, 128), lambda i: (0, i, 0)),
            pl.BlockSpec((1, d_out), lambda i: (0, 0)),
        ],
        out_specs=pl.BlockSpec((rb, d_out), lambda i: (i, 0)),
        out_shape=jax.ShapeDtypeStruct((n, d_out), jnp.float32),
    )(partials, y, cnt, bias.reshape(1, d_out))
    return out


# speedup vs baseline: 1.5569x; 1.5569x over previous
import functools

import jax
import jax.numpy as jnp
from jax import lax
from jax.experimental import pallas as pl
from jax.experimental.pallas import tpu as pltpu
from jax.experimental.pallas import tpu_sc as plsc

NC = 2
NS = 16
NW = NC * NS
CHUNK = 128


def _deg_body(dst_hbm, ones_hbm, z_hbm, cnt_hbm, idx_v, ones_v, acc_sh):
    cid = lax.axis_index("c")
    sid = lax.axis_index("s")
    wid = cid * NS + sid
    cpt = dst_hbm.shape[1]
    rpt = acc_sh.shape[0] // NS

    pltpu.sync_copy(dst_hbm.at[wid], idx_v)
    pltpu.sync_copy(ones_hbm, ones_v)
    pltpu.sync_copy(z_hbm, acc_sh.at[pl.ds(sid * rpt, rpt)])
    plsc.subcore_barrier()

    @pl.loop(0, cpt)
    def _(j):
        pltpu.sync_copy(ones_v, acc_sh.at[idx_v.at[j]], add=True)

    plsc.subcore_barrier()
    pltpu.sync_copy(acc_sh.at[pl.ds(sid * rpt, rpt)],
                    cnt_hbm.at[cid, pl.ds(sid * rpt, rpt)])


NBUF = 2


PACK_BITS = 15

CPT0 = 120
CPT1 = 38


def _scatter_body(y_hbm, pk_hbm, z_hbm, out_hbm,
                  pkv, srcu, dstu, rows_v, acc_sh, gsem0, gsem1):
    cid = lax.axis_index("c")
    sid = lax.axis_index("s")
    wid = cid * NS + sid
    rpt = acc_sh.shape[0] // NS
    gsems = (gsem0, gsem1)
    mask = jnp.int32((1 << PACK_BITS) - 1)
    cpt_c = jnp.where(cid == 0, CPT0, CPT1)

    pltpu.sync_copy(pk_hbm.at[wid], pkv)
    pltpu.sync_copy(z_hbm, acc_sh.at[pl.ds(sid * rpt, rpt)])

    def unpack(j, b):
        for k in range(CHUNK // 16):
            p16 = pkv[j, pl.ds(k * 16, 16)]
            srcu[b, pl.ds(k * 16, 16)] = lax.shift_right_logical(
                p16, PACK_BITS)
            dstu[b, pl.ds(k * 16, 16)] = lax.bitwise_and(p16, mask)

    plsc.subcore_barrier()

    for b in range(NBUF):
        unpack(b, b)
        pltpu.async_copy(y_hbm.at[srcu.at[b]], rows_v.at[b], gsems[b])

    @pl.loop(0, cpt_c, step=NBUF)
    def _(j0):
        for b in range(NBUF):
            j = j0 + b
            pltpu.make_async_copy(y_hbm.at[srcu.at[b]], rows_v.at[b],
                                  gsems[b]).wait()
            pltpu.sync_copy(rows_v.at[b], acc_sh.at[dstu.at[b]], add=True)
            jn = jnp.minimum(j + NBUF, cpt_c - 1)
            unpack(jn, b)
            pltpu.async_copy(y_hbm.at[srcu.at[b]], rows_v.at[b], gsems[b])

    for b in range(NBUF):
        pltpu.make_async_copy(y_hbm.at[srcu.at[b]], rows_v.at[b],
                              gsems[b]).wait()

    plsc.subcore_barrier()
    pltpu.sync_copy(acc_sh.at[pl.ds(sid * rpt, rpt)],
                    out_hbm.at[cid, pl.ds(sid * rpt, rpt)])


def _y_body(x_ref, w_ref, cnt_ref, y_ref):
    deg = cnt_ref[0, :, 0:1] + cnt_ref[1, :, 0:1] + 1.0
    dis = lax.rsqrt(deg)
    y_ref[...] = dis * jnp.dot(x_ref[...], w_ref[...],
                               preferred_element_type=jnp.float32)


def _combine_body(p_ref, y_ref, cnt_ref, b_ref, o_ref):
    deg = cnt_ref[0, :, 0:1] + cnt_ref[1, :, 0:1] + 1.0
    dis = lax.rsqrt(deg)
    o_ref[...] = dis * (p_ref[0] + p_ref[1] + y_ref[...]) + b_ref[...]


def kernel(x, edge_index, weight, bias):
    n, d_in = x.shape
    d_out = weight.shape[1]
    e = edge_index.shape[1]

    src = edge_index[0].astype(jnp.int32)
    dst = edge_index[1].astype(jnp.int32)

    cpt = -(-e // (NW * CHUNK))
    cpt = -(-cpt // NBUF) * NBUF
    e_pad = NW * cpt * CHUNK
    pad = e_pad - e
    dst_p = jnp.concatenate([dst, jnp.full((pad,), n, jnp.int32)])
    dst3 = dst_p.reshape(NW, cpt, CHUNK)

    rpt = -(-(n + 1) // (NS * 8)) * 8
    r = rpt * NS

    ones128 = jnp.ones((CHUNK, 128), jnp.float32)
    z128 = jnp.zeros((rpt, d_out), jnp.float32)

    mesh = plsc.VectorSubcoreMesh(core_axis_name="c", subcore_axis_name="s")

    deg_k = pl.kernel(
        _deg_body,
        out_type=jax.ShapeDtypeStruct((NC, r, 128), jnp.float32),
        mesh=mesh,
        scratch_types=[
            pltpu.VMEM((cpt, CHUNK), jnp.int32),
            pltpu.VMEM((CHUNK, 128), jnp.float32),
            pltpu.VMEM_SHARED((r, 128), jnp.float32),
        ],
    )
    cnt = deg_k(dst3, ones128, z128)

    rb = 400
    grid = n // rb
    y = pl.pallas_call(
        _y_body,
        grid=(grid,),
        in_specs=[
            pl.BlockSpec((rb, d_in), lambda i: (i, 0)),
            pl.BlockSpec((d_in, d_out), lambda i: (0, 0)),
            pl.BlockSpec((NC, rb, 128), lambda i: (0, i, 0)),
        ],
        out_specs=pl.BlockSpec((rb, d_out), lambda i: (i, 0)),
        out_shape=jax.ShapeDtypeStruct((n, d_out), jnp.float32),
    )(x, weight, cnt)

    e_pad2 = NS * CHUNK * (CPT0 + CPT1)
    pad2 = e_pad2 - e
    src_q = jnp.concatenate([src, jnp.zeros((pad2,), jnp.int32)])
    dst_q = jnp.concatenate([dst, jnp.full((pad2,), n, jnp.int32)])
    packed = (src_q << PACK_BITS) | dst_q
    cpt_max = max(CPT0, CPT1)
    p0 = packed[:NS * CPT0 * CHUNK].reshape(NS, CPT0, CHUNK)
    p1 = packed[NS * CPT0 * CHUNK:].reshape(NS, CPT1, CHUNK)
    p0 = jnp.pad(p0, ((0, 0), (0, cpt_max - CPT0), (0, 0)))
    p1 = jnp.pad(p1, ((0, 0), (0, cpt_max - CPT1), (0, 0)))
    packed3 = jnp.concatenate([p0, p1]).reshape(NW, cpt_max, CHUNK)

    scat_k = pl.kernel(
        _scatter_body,
        out_type=jax.ShapeDtypeStruct((NC, r, d_out), jnp.float32),
        mesh=mesh,
        scratch_types=[
            pltpu.VMEM((cpt_max, CHUNK), jnp.int32),
            pltpu.VMEM((NBUF, CHUNK), jnp.int32),
            pltpu.VMEM((NBUF, CHUNK), jnp.int32),
            pltpu.VMEM((NBUF, CHUNK, d_out), jnp.float32),
            pltpu.VMEM_SHARED((r, d_out), jnp.float32),
        ] + [pltpu.SemaphoreType.DMA] * NBUF,
    )
    partials = scat_k(y, packed3, z128)

    out = pl.pallas_call(
        _combine_body,
        grid=(grid,),
        in_specs=[
            pl.BlockSpec((NC, rb, d_out), lambda i: (0, i, 0)),
            pl.BlockSpec((rb, d_out), lambda i: (i, 0)),
            pl.BlockSpec((NC, rb, 128), lambda i: (0, i, 0)),
            pl.BlockSpec((1, d_out), lambda i: (0, 0)),
        ],
        out_specs=pl.BlockSpec((rb, d_out), lambda i: (i, 0)),
        out_shape=jax.ShapeDtypeStruct((n, d_out), jnp.float32),
    )(partials, y, cnt, bias.reshape(1, d_out))
    return out


# speedup vs baseline: 1.5739x; 1.0110x over previous
import functools

import jax
import jax.numpy as jnp
from jax import lax
from jax.experimental import pallas as pl
from jax.experimental.pallas import tpu as pltpu
from jax.experimental.pallas import tpu_sc as plsc

NC = 2
NS = 16
NW = NC * NS
CHUNK = 128


def _deg_body(dst_hbm, ones_hbm, z_hbm, cnt_hbm, idx_v, ones_v, acc_sh):
    cid = lax.axis_index("c")
    sid = lax.axis_index("s")
    wid = cid * NS + sid
    cpt = dst_hbm.shape[1]
    rpt = acc_sh.shape[0] // NS

    pltpu.sync_copy(dst_hbm.at[wid], idx_v)
    pltpu.sync_copy(ones_hbm, ones_v)
    pltpu.sync_copy(z_hbm, acc_sh.at[pl.ds(sid * rpt, rpt)])
    plsc.subcore_barrier()

    @pl.loop(0, cpt)
    def _(j):
        pltpu.sync_copy(ones_v, acc_sh.at[idx_v.at[j]], add=True)

    plsc.subcore_barrier()
    pltpu.sync_copy(acc_sh.at[pl.ds(sid * rpt, rpt)],
                    cnt_hbm.at[cid, pl.ds(sid * rpt, rpt)])


NBUF = 2


PACK_BITS = 15

CPT0 = 124
CPT1 = 34


def _scatter_body(y_hbm, pk_hbm, z_hbm, out_hbm,
                  pkv, srcu, dstu, rows_v, acc_sh, gsem0, gsem1):
    cid = lax.axis_index("c")
    sid = lax.axis_index("s")
    wid = cid * NS + sid
    rpt = acc_sh.shape[0] // NS
    gsems = (gsem0, gsem1)
    mask = jnp.int32((1 << PACK_BITS) - 1)
    cpt_c = jnp.where(cid == 0, CPT0, CPT1)

    pltpu.sync_copy(pk_hbm.at[wid], pkv)
    pltpu.sync_copy(z_hbm, acc_sh.at[pl.ds(sid * rpt, rpt)])

    def unpack(j, b):
        for k in range(CHUNK // 16):
            p16 = pkv[j, pl.ds(k * 16, 16)]
            srcu[b, pl.ds(k * 16, 16)] = lax.shift_right_logical(
                p16, PACK_BITS)
            dstu[b, pl.ds(k * 16, 16)] = lax.bitwise_and(p16, mask)

    plsc.subcore_barrier()

    for b in range(NBUF):
        unpack(b, b)
        pltpu.async_copy(y_hbm.at[srcu.at[b]], rows_v.at[b], gsems[b])

    @pl.loop(0, cpt_c, step=NBUF)
    def _(j0):
        for b in range(NBUF):
            j = j0 + b
            pltpu.make_async_copy(y_hbm.at[srcu.at[b]], rows_v.at[b],
                                  gsems[b]).wait()
            pltpu.sync_copy(rows_v.at[b], acc_sh.at[dstu.at[b]], add=True)
            jn = jnp.minimum(j + NBUF, cpt_c - 1)
            unpack(jn, b)
            pltpu.async_copy(y_hbm.at[srcu.at[b]], rows_v.at[b], gsems[b])

    for b in range(NBUF):
        pltpu.make_async_copy(y_hbm.at[srcu.at[b]], rows_v.at[b],
                              gsems[b]).wait()

    plsc.subcore_barrier()
    pltpu.sync_copy(acc_sh.at[pl.ds(sid * rpt, rpt)],
                    out_hbm.at[cid, pl.ds(sid * rpt, rpt)])


def _y_body(x_ref, w_ref, cnt_ref, y_ref):
    deg = cnt_ref[0, :, 0:1] + cnt_ref[1, :, 0:1] + 1.0
    dis = lax.rsqrt(deg)
    y_ref[...] = dis * jnp.dot(x_ref[...], w_ref[...],
                               preferred_element_type=jnp.float32)


def _combine_body(p_ref, y_ref, cnt_ref, b_ref, o_ref):
    deg = cnt_ref[0, :, 0:1] + cnt_ref[1, :, 0:1] + 1.0
    dis = lax.rsqrt(deg)
    o_ref[...] = dis * (p_ref[0] + p_ref[1] + y_ref[...]) + b_ref[...]


def kernel(x, edge_index, weight, bias):
    n, d_in = x.shape
    d_out = weight.shape[1]
    e = edge_index.shape[1]

    src = edge_index[0].astype(jnp.int32)
    dst = edge_index[1].astype(jnp.int32)

    cpt = -(-e // (NW * CHUNK))
    cpt = -(-cpt // NBUF) * NBUF
    e_pad = NW * cpt * CHUNK
    pad = e_pad - e
    dst_p = jnp.concatenate([dst, jnp.full((pad,), n, jnp.int32)])
    dst3 = dst_p.reshape(NW, cpt, CHUNK)

    rpt = -(-(n + 1) // (NS * 8)) * 8
    r = rpt * NS

    ones128 = jnp.ones((CHUNK, 128), jnp.float32)
    z128 = jnp.zeros((rpt, d_out), jnp.float32)

    mesh = plsc.VectorSubcoreMesh(core_axis_name="c", subcore_axis_name="s")

    deg_k = pl.kernel(
        _deg_body,
        out_type=jax.ShapeDtypeStruct((NC, r, 128), jnp.float32),
        mesh=mesh,
        scratch_types=[
            pltpu.VMEM((cpt, CHUNK), jnp.int32),
            pltpu.VMEM((CHUNK, 128), jnp.float32),
            pltpu.VMEM_SHARED((r, 128), jnp.float32),
        ],
    )
    cnt = deg_k(dst3, ones128, z128)

    rb = 400
    grid = n // rb
    y = pl.pallas_call(
        _y_body,
        grid=(grid,),
        in_specs=[
            pl.BlockSpec((rb, d_in), lambda i: (i, 0)),
            pl.BlockSpec((d_in, d_out), lambda i: (0, 0)),
            pl.BlockSpec((NC, rb, 128), lambda i: (0, i, 0)),
        ],
        out_specs=pl.BlockSpec((rb, d_out), lambda i: (i, 0)),
        out_shape=jax.ShapeDtypeStruct((n, d_out), jnp.float32),
    )(x, weight, cnt)

    e_pad2 = NS * CHUNK * (CPT0 + CPT1)
    pad2 = e_pad2 - e
    src_q = jnp.concatenate([src, jnp.zeros((pad2,), jnp.int32)])
    dst_q = jnp.concatenate([dst, jnp.full((pad2,), n, jnp.int32)])
    packed = (src_q << PACK_BITS) | dst_q
    cpt_max = max(CPT0, CPT1)
    p0 = packed[:NS * CPT0 * CHUNK].reshape(NS, CPT0, CHUNK)
    p1 = packed[NS * CPT0 * CHUNK:].reshape(NS, CPT1, CHUNK)
    p0 = jnp.pad(p0, ((0, 0), (0, cpt_max - CPT0), (0, 0)))
    p1 = jnp.pad(p1, ((0, 0), (0, cpt_max - CPT1), (0, 0)))
    packed3 = jnp.concatenate([p0, p1]).reshape(NW, cpt_max, CHUNK)

    scat_k = pl.kernel(
        _scatter_body,
        out_type=jax.ShapeDtypeStruct((NC, r, d_out), jnp.float32),
        mesh=mesh,
        scratch_types=[
            pltpu.VMEM((cpt_max, CHUNK), jnp.int32),
            pltpu.VMEM((NBUF, CHUNK), jnp.int32),
            pltpu.VMEM((NBUF, CHUNK), jnp.int32),
            pltpu.VMEM((NBUF, CHUNK, d_out), jnp.float32),
            pltpu.VMEM_SHARED((r, d_out), jnp.float32),
        ] + [pltpu.SemaphoreType.DMA] * NBUF,
    )
    partials = scat_k(y, packed3, z128)

    out = pl.pallas_call(
        _combine_body,
        grid=(grid,),
        in_specs=[
            pl.BlockSpec((NC, rb, d_out), lambda i: (0, i, 0)),
            pl.BlockSpec((rb, d_out), lambda i: (i, 0)),
            pl.BlockSpec((NC, rb, 128), lambda i: (0, i, 0)),
            pl.BlockSpec((1, d_out), lambda i: (0, 0)),
        ],
        out_specs=pl.BlockSpec((rb, d_out), lambda i: (i, 0)),
        out_shape=jax.ShapeDtypeStruct((n, d_out), jnp.float32),
    )(partials, y, cnt, bias.reshape(1, d_out))
    return out


# speedup vs baseline: 1.7180x; 1.0916x over previous
import functools

import jax
import jax.numpy as jnp
from jax import lax
from jax.experimental import pallas as pl
from jax.experimental.pallas import tpu as pltpu
from jax.experimental.pallas import tpu_sc as plsc

NC = 2
NS = 16
NW = NC * NS
CHUNK = 128


def _deg_body(dst_hbm, ones_hbm, z_hbm, cnt_hbm, idx_v, ones_v, acc_sh,
              dsem):
    cid = lax.axis_index("c")
    sid = lax.axis_index("s")
    wid = cid * NS + sid
    cpt = dst_hbm.shape[1]
    rpt = acc_sh.shape[0] // NS

    pltpu.sync_copy(dst_hbm.at[wid], idx_v)
    pltpu.sync_copy(ones_hbm, ones_v)
    pltpu.sync_copy(z_hbm, acc_sh.at[pl.ds(sid * rpt, rpt)])
    plsc.subcore_barrier()

    @pl.loop(0, cpt, step=4)
    def _(j0):
        for b in range(4):
            pltpu.async_copy(ones_v, acc_sh.at[idx_v.at[j0 + b]], dsem,
                             add=True)
        for b in range(4):
            pltpu.make_async_copy(ones_v, acc_sh.at[idx_v.at[j0 + b]],
                                  dsem).wait()

    plsc.subcore_barrier()
    pltpu.sync_copy(acc_sh.at[pl.ds(sid * rpt, rpt)],
                    cnt_hbm.at[cid, pl.ds(sid * rpt, rpt)])


NBUF = 2


PACK_BITS = 15

CPT0 = 124
CPT1 = 34


def _scatter_body(y_hbm, pk_hbm, z_hbm, out_hbm,
                  pkv, srcu, dstu, rows_v, acc_sh, gsem0, gsem1):
    cid = lax.axis_index("c")
    sid = lax.axis_index("s")
    wid = cid * NS + sid
    rpt = acc_sh.shape[0] // NS
    gsems = (gsem0, gsem1)
    mask = jnp.int32((1 << PACK_BITS) - 1)
    cpt_c = jnp.where(cid == 0, CPT0, CPT1)

    pltpu.sync_copy(pk_hbm.at[wid], pkv)

    def unpack(j, b):
        for k in range(CHUNK // 16):
            p16 = pkv[j, pl.ds(k * 16, 16)]
            srcu[b, pl.ds(k * 16, 16)] = lax.shift_right_logical(
                p16, PACK_BITS)
            dstu[b, pl.ds(k * 16, 16)] = lax.bitwise_and(p16, mask)

    for b in range(NBUF):
        unpack(b, b)
        pltpu.async_copy(y_hbm.at[srcu.at[b]], rows_v.at[b], gsems[b])

    pltpu.sync_copy(z_hbm, acc_sh.at[pl.ds(sid * rpt, rpt)])
    plsc.subcore_barrier()

    @pl.loop(0, cpt_c, step=NBUF)
    def _(j0):
        for b in range(NBUF):
            j = j0 + b
            pltpu.make_async_copy(y_hbm.at[srcu.at[b]], rows_v.at[b],
                                  gsems[b]).wait()
            pltpu.sync_copy(rows_v.at[b], acc_sh.at[dstu.at[b]], add=True)
            jn = jnp.minimum(j + NBUF, cpt_c - 1)
            unpack(jn, b)
            pltpu.async_copy(y_hbm.at[srcu.at[b]], rows_v.at[b], gsems[b])

    for b in range(NBUF):
        pltpu.make_async_copy(y_hbm.at[srcu.at[b]], rows_v.at[b],
                              gsems[b]).wait()

    plsc.subcore_barrier()
    pltpu.sync_copy(acc_sh.at[pl.ds(sid * rpt, rpt)],
                    out_hbm.at[cid, pl.ds(sid * rpt, rpt)])


def _y_body(x_ref, w_ref, cnt_ref, y_ref, dis_ref):
    deg = cnt_ref[0, :, 0:1] + cnt_ref[1, :, 0:1] + 1.0
    dis = lax.rsqrt(deg)
    y_ref[...] = dis * jnp.dot(x_ref[...], w_ref[...],
                               preferred_element_type=jnp.float32)
    dis_ref[...] = jnp.broadcast_to(dis, dis_ref.shape)


def _combine_body(p_ref, y_ref, dis_ref, b_ref, o_ref):
    dis = dis_ref[:, 0:1]
    o_ref[...] = dis * (p_ref[0] + p_ref[1] + y_ref[...]) + b_ref[...]


def kernel(x, edge_index, weight, bias):
    n, d_in = x.shape
    d_out = weight.shape[1]
    e = edge_index.shape[1]

    src = edge_index[0].astype(jnp.int32)
    dst = edge_index[1].astype(jnp.int32)

    cpt = -(-e // (NW * CHUNK))
    cpt = -(-cpt // NBUF) * NBUF
    e_pad = NW * cpt * CHUNK
    pad = e_pad - e
    dst_p = jnp.concatenate([dst, jnp.full((pad,), n, jnp.int32)])
    dst3 = dst_p.reshape(NW, cpt, CHUNK)

    rpt = -(-(n + 1) // (NS * 8)) * 8
    r = rpt * NS

    ones128 = jnp.ones((CHUNK, 128), jnp.float32)
    z128 = jnp.zeros((rpt, d_out), jnp.float32)

    mesh = plsc.VectorSubcoreMesh(core_axis_name="c", subcore_axis_name="s")

    deg_k = pl.kernel(
        _deg_body,
        out_type=jax.ShapeDtypeStruct((NC, r, 128), jnp.float32),
        mesh=mesh,
        scratch_types=[
            pltpu.VMEM((cpt, CHUNK), jnp.int32),
            pltpu.VMEM((CHUNK, 128), jnp.float32),
            pltpu.VMEM_SHARED((r, 128), jnp.float32),
            pltpu.SemaphoreType.DMA,
        ],
    )
    cnt = deg_k(dst3, ones128, z128)

    rb = 400
    grid = n // rb
    y, dis = pl.pallas_call(
        _y_body,
        grid=(grid,),
        in_specs=[
            pl.BlockSpec((rb, d_in), lambda i: (i, 0)),
            pl.BlockSpec((d_in, d_out), lambda i: (0, 0)),
            pl.BlockSpec((NC, rb, 128), lambda i: (0, i, 0)),
        ],
        out_specs=[pl.BlockSpec((rb, d_out), lambda i: (i, 0)),
                   pl.BlockSpec((rb, 8), lambda i: (i, 0))],
        out_shape=[jax.ShapeDtypeStruct((n, d_out), jnp.float32),
                   jax.ShapeDtypeStruct((n, 8), jnp.float32)],
    )(x, weight, cnt)

    e_pad2 = NS * CHUNK * (CPT0 + CPT1)
    pad2 = e_pad2 - e
    src_q = jnp.concatenate([src, jnp.zeros((pad2,), jnp.int32)])
    dst_q = jnp.concatenate([dst, jnp.full((pad2,), n, jnp.int32)])
    packed = (src_q << PACK_BITS) | dst_q
    cpt_max = max(CPT0, CPT1)
    p0 = packed[:NS * CPT0 * CHUNK].reshape(NS, CPT0, CHUNK)
    p1 = packed[NS * CPT0 * CHUNK:].reshape(NS, CPT1, CHUNK)
    p0 = jnp.pad(p0, ((0, 0), (0, cpt_max - CPT0), (0, 0)))
    p1 = jnp.pad(p1, ((0, 0), (0, cpt_max - CPT1), (0, 0)))
    packed3 = jnp.concatenate([p0, p1]).reshape(NW, cpt_max, CHUNK)

    scat_k = pl.kernel(
        _scatter_body,
        out_type=jax.ShapeDtypeStruct((NC, r, d_out), jnp.float32),
        mesh=mesh,
        scratch_types=[
            pltpu.VMEM((cpt_max, CHUNK), jnp.int32),
            pltpu.VMEM((NBUF, CHUNK), jnp.int32),
            pltpu.VMEM((NBUF, CHUNK), jnp.int32),
            pltpu.VMEM((NBUF, CHUNK, d_out), jnp.float32),
            pltpu.VMEM_SHARED((r, d_out), jnp.float32),
        ] + [pltpu.SemaphoreType.DMA] * NBUF,
    )
    partials = scat_k(y, packed3, z128)

    out = pl.pallas_call(
        _combine_body,
        grid=(grid,),
        in_specs=[
            pl.BlockSpec((NC, rb, d_out), lambda i: (0, i, 0)),
            pl.BlockSpec((rb, d_out), lambda i: (i, 0)),
            pl.BlockSpec((rb, 8), lambda i: (i, 0)),
            pl.BlockSpec((1, d_out), lambda i: (0, 0)),
        ],
        out_specs=pl.BlockSpec((rb, d_out), lambda i: (i, 0)),
        out_shape=jax.ShapeDtypeStruct((n, d_out), jnp.float32),
    )(partials, y, dis, bias.reshape(1, d_out))
    return out
